# baseline (speedup 1.0000x reference)
import functools

import jax
import jax.numpy as jnp
from jax import lax
from jax.experimental import pallas as pl
from jax.experimental.pallas import tpu as pltpu
from jax.experimental.pallas import tpu_sc as plsc

_NUM_EMBEDDINGS = 1000000
_DIM = 64
_B, _S = 16384, 50
_TOTAL = _B * _S
_NC, _NS = 2, 16
_NW = _NC * _NS
_G = 128
_NGW = _TOTAL // _G // _NW
_KB = 4
_NSUP = _NGW // _KB
_NPAIR = _NSUP // 2
_LANES = 16
_DB = _DIM // 8
_NB = _B // _G

_mesh = plsc.VectorSubcoreMesh(core_axis_name="c", subcore_axis_name="s")


@functools.partial(
    pl.kernel,
    mesh=_mesh,
    out_type=jax.ShapeDtypeStruct((_S, _DB, _NB, 8, _G), jnp.float32),
    scratch_types=[
        pltpu.VMEM((_NGW, _G), jnp.int32),
        pltpu.VMEM((2, _KB, _G, _DIM), jnp.float32),
        pltpu.VMEM((_KB, _DB, 8, _G), jnp.float32),
        pltpu.SemaphoreType.DMA,
        pltpu.SemaphoreType.DMA,
        pltpu.SemaphoreType.DMA,
        pltpu.SemaphoreType.DMA,
        pltpu.SemaphoreType.DMA,
        pltpu.SemaphoreType.DMA,
    ],
    compiler_params=pltpu.CompilerParams(
        use_tc_tiling_on_sc=False, needs_layout_passes=False
    ),
)
def _sc_embedding_lookup(idx_hbm, table_hbm, out_hbm, idx_v, rows_v, tile_v,
                         gsem0, gsem1, ssem0, ssem1, ssem2, ssem3):
    ssems = (ssem0, ssem1, ssem2, ssem3)
    wid = lax.axis_index("s") * _NC + lax.axis_index("c")
    gbase = wid * _NGW

    pltpu.sync_copy(idx_hbm.at[wid], idx_v)

    iota16 = lax.iota(jnp.int32, 16)
    l_vecs = [iota16 + 16 * lc for lc in range(_G // _LANES)]

    def clamp_chunk(c):
        for j in range(_KB):
            for i in range(_G // _LANES):
                sl = pl.ds(i * _LANES, _LANES)
                v = idx_v[c * _KB + j, sl]
                idx_v[c * _KB + j, sl] = jnp.minimum(
                    jnp.maximum(v, 0), _NUM_EMBEDDINGS - 1
                )

    def fire_gathers(c, h, gsem):
        clamp_chunk(c)
        for j in range(_KB):
            pltpu.async_copy(
                table_hbm.at[idx_v.at[c * _KB + j]], rows_v.at[h, j], gsem
            )

    def drain_gathers(c, h, gsem):
        for j in range(_KB):
            pltpu.make_async_copy(
                table_hbm.at[idx_v.at[c * _KB + j]], rows_v.at[h, j], gsem
            ).wait()

    def drain_one_store_group(sem):
        pltpu.make_async_copy(
            table_hbm.at[pl.ds(0, _G)], rows_v.at[0, 0], sem
        ).wait()

    def process_chunk(c, h):
        for j in range(_KB):
            m = c * _KB + j
            gl = gbase + m
            sg = gl // _NB
            bb = gl % _NB
            sem = ssems[j]

            @pl.when(c >= 1)
            def _():
                drain_one_store_group(sem)

            def per_block(a, _):
                for cd in range(8):
                    col = jnp.full((_LANES,), 0, jnp.int32) + (8 * a + cd)
                    for lc in range(_G // _LANES):
                        v = plsc.load_gather(
                            rows_v.at[h, j], [l_vecs[lc], col]
                        )
                        tile_v[j, a, cd, pl.ds(16 * lc, _LANES)] = v
                pltpu.async_copy(
                    tile_v.at[j].at[a], out_hbm.at[sg].at[a].at[bb], sem
                )
                return 0

            lax.fori_loop(0, _DB, per_block, 0)

    fire_gathers(0, 0, gsem0)

    def pair(t, _):
        c0 = 2 * t
        c1 = 2 * t + 1
        fire_gathers(c1, 1, gsem1)
        drain_gathers(c0, 0, gsem0)
        process_chunk(c0, 0)

        @pl.when(t + 1 < _NPAIR)
        def _():
            fire_gathers(c0 + 2, 0, gsem0)

        drain_gathers(c1, 1, gsem1)
        process_chunk(c1, 1)
        return 0

    lax.fori_loop(0, _NPAIR, pair, 0)

    for j in range(_KB):
        drain_one_store_group(ssems[j])


def kernel(idx, table):
    flat = idx.T.reshape(_NW, _NGW, _G).astype(jnp.int32)
    out = _sc_embedding_lookup(flat, table)
    return out.transpose(2, 4, 0, 1, 3).reshape(_B, _S, _DIM)


# speedup vs baseline: 1.5198x; 1.5198x over previous
import functools

import jax
import jax.numpy as jnp
from jax import lax
from jax.experimental import pallas as pl
from jax.experimental.pallas import tpu as pltpu
from jax.experimental.pallas import tpu_sc as plsc

_NUM_EMBEDDINGS = 1000000
_DIM = 64
_B, _S = 16384, 50
_TOTAL = _B * _S
_NC, _NS = 2, 16
_NW = _NC * _NS
_G = 128
_NGW = _TOTAL // _G // _NW
_KB = 4
_NSUP = _NGW // _KB
_NPAIR = _NSUP // 2
_LANES = 16
_DB = _DIM // 8
_NB = _B // _G

_mesh = plsc.VectorSubcoreMesh(core_axis_name="c", subcore_axis_name="s")


@functools.partial(
    pl.kernel,
    mesh=_mesh,
    out_type=jax.ShapeDtypeStruct((_S, _DB, _NB, 8, _G), jnp.float32),
    scratch_types=[
        pltpu.VMEM((_NGW, _G), jnp.int32),
        pltpu.VMEM((2, _KB, _G, _DIM), jnp.float32),
        pltpu.VMEM((_KB, _DB, 8, _G), jnp.float32),
        pltpu.SemaphoreType.DMA,
        pltpu.SemaphoreType.DMA,
        pltpu.SemaphoreType.DMA,
        pltpu.SemaphoreType.DMA,
        pltpu.SemaphoreType.DMA,
        pltpu.SemaphoreType.DMA,
    ],
    compiler_params=pltpu.CompilerParams(
        use_tc_tiling_on_sc=False, needs_layout_passes=False
    ),
)
def _sc_embedding_lookup(idx_hbm, table_hbm, out_hbm, idx_v, rows_v, tile_v,
                         gsem0, gsem1, ssem0, ssem1, ssem2, ssem3):
    ssems = (ssem0, ssem1, ssem2, ssem3)
    wid = lax.axis_index("s") * _NC + lax.axis_index("c")
    gbase = wid * _NGW

    pltpu.sync_copy(idx_hbm.at[wid], idx_v)

    iota16 = lax.iota(jnp.int32, 16)
    rots = [lax.rem(iota16 + k, 16) for k in range(_LANES)]
    rot_as = [lax.shift_right_logical(r, 3) for r in rots]
    rot_cs = [lax.bitwise_and(r, 7) for r in rots]

    def clamp_chunk(c):
        for j in range(_KB):
            for i in range(_G // _LANES):
                sl = pl.ds(i * _LANES, _LANES)
                v = idx_v[c * _KB + j, sl]
                idx_v[c * _KB + j, sl] = jnp.minimum(
                    jnp.maximum(v, 0), _NUM_EMBEDDINGS - 1
                )

    def fire_gathers(c, h, gsem):
        clamp_chunk(c)
        for j in range(_KB):
            pltpu.async_copy(
                table_hbm.at[idx_v.at[c * _KB + j]], rows_v.at[h, j], gsem
            )

    def drain_gathers(c, h, gsem):
        for j in range(_KB):
            pltpu.make_async_copy(
                table_hbm.at[idx_v.at[c * _KB + j]], rows_v.at[h, j], gsem
            ).wait()

    def drain_one_store_group(sem):
        pltpu.make_async_copy(
            table_hbm.at[pl.ds(0, _G)], rows_v.at[0, 0], sem
        ).wait()

    def process_chunk(c, h):
        for j in range(_KB):
            m = c * _KB + j
            gl = gbase + m
            sg = gl // _NB
            bb = gl % _NB
            sem = ssems[j]

            @pl.when(c >= 1)
            def _():
                drain_one_store_group(sem)

            def sub_l(lb, _):
                l_vec = iota16 + 16 * lb
                for sc in range(_DIM // _LANES):
                    for k in range(_LANES):
                        v = plsc.load_gather(
                            rows_v.at[h, j], [l_vec, rots[k] + 16 * sc]
                        )
                        plsc.store_scatter(
                            tile_v.at[j],
                            [rot_as[k] + 2 * sc, rot_cs[k], l_vec],
                            v,
                        )
                return 0

            lax.fori_loop(0, _G // _LANES, sub_l, 0)
            for a in range(_DB):
                pltpu.async_copy(
                    tile_v.at[j].at[a], out_hbm.at[sg].at[a].at[bb], sem
                )

    fire_gathers(0, 0, gsem0)

    def pair(t, _):
        c0 = 2 * t
        c1 = 2 * t + 1
        fire_gathers(c1, 1, gsem1)
        drain_gathers(c0, 0, gsem0)
        process_chunk(c0, 0)

        @pl.when(t + 1 < _NPAIR)
        def _():
            fire_gathers(c0 + 2, 0, gsem0)

        drain_gathers(c1, 1, gsem1)
        process_chunk(c1, 1)
        return 0

    lax.fori_loop(0, _NPAIR, pair, 0)

    for j in range(_KB):
        drain_one_store_group(ssems[j])


def kernel(idx, table):
    flat = idx.T.reshape(_NW, _NGW, _G).astype(jnp.int32)
    out = _sc_embedding_lookup(flat, table)
    return out.transpose(2, 4, 0, 1, 3).reshape(_B, _S, _DIM)


# speedup vs baseline: 1.8111x; 1.1916x over previous
import functools

import jax
import jax.numpy as jnp
from jax import lax
from jax.experimental import pallas as pl
from jax.experimental.pallas import tpu as pltpu
from jax.experimental.pallas import tpu_sc as plsc

_NUM_EMBEDDINGS = 1000000
_DIM = 64
_B, _S = 16384, 50
_TOTAL = _B * _S
_NC, _NS = 2, 16
_NW = _NC * _NS
_G = 128
_NGW = _TOTAL // _G // _NW
_KB = 4
_NSUP = _NGW // _KB
_NPAIR = _NSUP // 2
_LANES = 16
_DB = _DIM // 8
_NB = _B // _G

_mesh = plsc.VectorSubcoreMesh(core_axis_name="c", subcore_axis_name="s")


@functools.partial(
    pl.kernel,
    mesh=_mesh,
    out_type=jax.ShapeDtypeStruct((_S, _DB, _NB, 8, _G), jnp.float32),
    scratch_types=[
        pltpu.VMEM((_NGW, _G), jnp.int32),
        pltpu.VMEM((2, _KB, _G, _DIM), jnp.float32),
        pltpu.VMEM((_KB, _DB, 8, _G), jnp.float32),
        pltpu.SemaphoreType.DMA,
        pltpu.SemaphoreType.DMA,
        pltpu.SemaphoreType.DMA,
        pltpu.SemaphoreType.DMA,
        pltpu.SemaphoreType.DMA,
        pltpu.SemaphoreType.DMA,
    ],
    compiler_params=pltpu.CompilerParams(
        use_tc_tiling_on_sc=False, needs_layout_passes=False
    ),
)
def _sc_embedding_lookup(idx_hbm, table_hbm, out_hbm, idx_v, rows_v, tile_v,
                         gsem0, gsem1, ssem0, ssem1, ssem2, ssem3):
    ssems = (ssem0, ssem1, ssem2, ssem3)
    wid = lax.axis_index("s") * _NC + lax.axis_index("c")
    gbase = wid * _NGW

    pltpu.sync_copy(idx_hbm.at[wid], idx_v)

    iota16 = lax.iota(jnp.int32, 16)
    l_vecs = [iota16 + 16 * lb for lb in range(_G // _LANES)]

    def clamp_chunk(c):
        for j in range(_KB):
            for i in range(_G // _LANES):
                sl = pl.ds(i * _LANES, _LANES)
                v = idx_v[c * _KB + j, sl]
                idx_v[c * _KB + j, sl] = jnp.minimum(
                    jnp.maximum(v, 0), _NUM_EMBEDDINGS - 1
                )

    def fire_gathers(c, h, gsem):
        clamp_chunk(c)
        for j in range(_KB):
            pltpu.async_copy(
                table_hbm.at[idx_v.at[c * _KB + j]], rows_v.at[h, j], gsem
            )

    def drain_gathers(c, h, gsem):
        for j in range(_KB):
            pltpu.make_async_copy(
                table_hbm.at[idx_v.at[c * _KB + j]], rows_v.at[h, j], gsem
            ).wait()

    def drain_one_store_group(sem):
        pltpu.make_async_copy(
            table_hbm.at[pl.ds(0, _G)], rows_v.at[0, 0], sem
        ).wait()

    def process_chunk(c, h):
        for j in range(_KB):
            m = c * _KB + j
            gl = gbase + m
            sg = gl // _NB
            bb = gl % _NB
            sem = ssems[j]

            @pl.when(c >= 1)
            def _():
                drain_one_store_group(sem)

            def shift_k(k, _):
                rot = lax.rem(iota16 + k, 16)
                cols = [rot + 16 * sc for sc in range(_DIM // _LANES)]
                a_idx = [
                    lax.shift_right_logical(rot, 3) + 2 * sc
                    for sc in range(_DIM // _LANES)
                ]
                c_idx = lax.bitwise_and(rot, 7)
                for lb in range(_G // _LANES):
                    for sc in range(_DIM // _LANES):
                        v = plsc.load_gather(
                            rows_v.at[h, j], [l_vecs[lb], cols[sc]]
                        )
                        plsc.store_scatter(
                            tile_v.at[j],
                            [a_idx[sc], c_idx, l_vecs[lb]],
                            v,
                        )
                return 0

            lax.fori_loop(0, _LANES, shift_k, 0)
            for a in range(_DB):
                pltpu.async_copy(
                    tile_v.at[j].at[a], out_hbm.at[sg].at[a].at[bb], sem
                )

    fire_gathers(0, 0, gsem0)

    def pair(t, _):
        c0 = 2 * t
        c1 = 2 * t + 1
        fire_gathers(c1, 1, gsem1)
        drain_gathers(c0, 0, gsem0)
        process_chunk(c0, 0)

        @pl.when(t + 1 < _NPAIR)
        def _():
            fire_gathers(c0 + 2, 0, gsem0)

        drain_gathers(c1, 1, gsem1)
        process_chunk(c1, 1)
        return 0

    lax.fori_loop(0, _NPAIR, pair, 0)

    for j in range(_KB):
        drain_one_store_group(ssems[j])


def kernel(idx, table):
    flat = idx.T.reshape(_NW, _NGW, _G).astype(jnp.int32)
    out = _sc_embedding_lookup(flat, table)
    return out.transpose(2, 4, 0, 1, 3).reshape(_B, _S, _DIM)


# speedup vs baseline: 1.9802x; 1.0934x over previous
import functools

import jax
import jax.numpy as jnp
from jax import lax
from jax.experimental import pallas as pl
from jax.experimental.pallas import tpu as pltpu
from jax.experimental.pallas import tpu_sc as plsc

_NUM_EMBEDDINGS = 1000000
_DIM = 64
_B, _S = 16384, 50
_TOTAL = _B * _S
_NC, _NS = 2, 16
_NW = _NC * _NS
_G = 128
_NGW = _TOTAL // _G // _NW
_KB = 2
_NSUP = _NGW // _KB
_NPAIR = _NSUP // 2
_LANES = 16
_DB = _DIM // 8
_NB = _B // _G

_mesh = plsc.VectorSubcoreMesh(core_axis_name="c", subcore_axis_name="s")


@functools.partial(
    pl.kernel,
    mesh=_mesh,
    out_type=jax.ShapeDtypeStruct((_S, _DB, _NB, 8, _G), jnp.float32),
    scratch_types=[
        pltpu.VMEM((_NGW, _G), jnp.int32),
        pltpu.VMEM((2, _KB, _G, 2 * _DIM), jnp.float32),
        pltpu.VMEM((_KB, _DB, 8, _G), jnp.float32),
        pltpu.SemaphoreType.DMA,
        pltpu.SemaphoreType.DMA,
        pltpu.SemaphoreType.DMA,
        pltpu.SemaphoreType.DMA,
    ],
    compiler_params=pltpu.CompilerParams(
        use_tc_tiling_on_sc=False, needs_layout_passes=False
    ),
)
def _sc_embedding_lookup(idx_hbm, table_hbm, out_hbm, idx_v, rows_v, tile_v,
                         gsem0, gsem1, ssem0, ssem1):
    ssems = (ssem0, ssem1)
    wid = lax.axis_index("s") * _NC + lax.axis_index("c")
    gbase = wid * _NGW

    pltpu.sync_copy(idx_hbm.at[wid], idx_v)

    iota16 = lax.iota(jnp.int32, 16)
    l_vecs = [iota16 + 16 * lb for lb in range(_G // _LANES)]

    def clamp_chunk(c):
        for j in range(_KB):
            for i in range(_G // _LANES):
                sl = pl.ds(i * _LANES, _LANES)
                v = idx_v[c * _KB + j, sl]
                idx_v[c * _KB + j, sl] = jnp.minimum(
                    jnp.maximum(v, 0), _NUM_EMBEDDINGS - 1
                )

    def fire_gathers(c, h, gsem):
        clamp_chunk(c)
        for j in range(_KB):
            pltpu.async_copy(
                table_hbm.at[idx_v.at[c * _KB + j]], rows_v.at[h, j], gsem
            )

    def drain_gathers(c, h, gsem):
        for j in range(_KB):
            pltpu.make_async_copy(
                table_hbm.at[idx_v.at[c * _KB + j]], rows_v.at[h, j], gsem
            ).wait()

    def drain_one_store_group(sem):
        pltpu.make_async_copy(
            out_hbm.at[0, 0, pl.ds(0, _DB)], tile_v.at[0], sem
        ).wait()

    def process_chunk(c, h):
        for j in range(_KB):
            m = c * _KB + j
            gl = gbase + m
            sg = gl // _NB
            bb = gl % _NB
            sem = ssems[j]

            @pl.when(c >= 1)
            def _():
                drain_one_store_group(sem)

            def shift_k(k, _):
                rot = lax.rem(iota16 + k, 16)
                cols = [rot + 16 * sc for sc in range(_DIM // _LANES)]
                a_idx = [
                    lax.shift_right_logical(rot, 3) + 2 * sc
                    for sc in range(_DIM // _LANES)
                ]
                c_idx = lax.bitwise_and(rot, 7)
                for lb in range(_G // _LANES):
                    for sc in range(_DIM // _LANES):
                        v = plsc.load_gather(
                            rows_v.at[h, j], [l_vecs[lb], cols[sc]]
                        )
                        plsc.store_scatter(
                            tile_v.at[j],
                            [a_idx[sc], c_idx, l_vecs[lb]],
                            v,
                        )
                return 0

            lax.fori_loop(0, _LANES, shift_k, 0)
            for a in range(_DB):
                pltpu.async_copy(
                    tile_v.at[j].at[a], out_hbm.at[sg].at[a].at[bb], sem
                )

    fire_gathers(0, 0, gsem0)

    def pair(t, _):
        c0 = 2 * t
        c1 = 2 * t + 1
        fire_gathers(c1, 1, gsem1)
        drain_gathers(c0, 0, gsem0)
        process_chunk(c0, 0)

        @pl.when(t + 1 < _NPAIR)
        def _():
            fire_gathers(c0 + 2, 0, gsem0)

        drain_gathers(c1, 1, gsem1)
        process_chunk(c1, 1)
        return 0

    lax.fori_loop(0, _NPAIR, pair, 0)

    for j in range(_KB):
        drain_one_store_group(ssems[j])


def kernel(idx, table):
    flat = idx.T.reshape(_NW, _NGW, _G).astype(jnp.int32)
    tab_pad = jnp.pad(table, ((0, 0), (0, _DIM)))
    out = _sc_embedding_lookup(flat, tab_pad)
    return out.transpose(2, 4, 0, 1, 3).reshape(_B, _S, _DIM)


# speedup vs baseline: 2.3918x; 1.2079x over previous
import functools

import jax
import jax.numpy as jnp
from jax import lax
from jax.experimental import pallas as pl
from jax.experimental.pallas import tpu as pltpu
from jax.experimental.pallas import tpu_sc as plsc

_NUM_EMBEDDINGS = 1000000
_DIM = 64
_B, _S = 16384, 50
_TOTAL = _B * _S
_NC, _NS = 2, 16
_NW = _NC * _NS
_G = 128
_NGW = _TOTAL // _G // _NW
_KB = 2
_NSUP = _NGW // _KB
_NPAIR = _NSUP // 2
_LANES = 16
_DB = _DIM // 8
_NB = _B // _G

_mesh = plsc.VectorSubcoreMesh(core_axis_name="c", subcore_axis_name="s")


@functools.partial(
    pl.kernel,
    mesh=_mesh,
    out_type=jax.ShapeDtypeStruct((_S, _DB, _NB, 8, _G), jnp.float32),
    scratch_types=[
        pltpu.VMEM((_NGW, _G), jnp.int32),
        pltpu.VMEM((2, _KB, _G, 2 * _DIM), jnp.float32),
        pltpu.VMEM((_KB, _DB, 8, _G), jnp.float32),
        pltpu.SemaphoreType.DMA,
        pltpu.SemaphoreType.DMA,
        pltpu.SemaphoreType.DMA,
        pltpu.SemaphoreType.DMA,
    ],
    compiler_params=pltpu.CompilerParams(
        use_tc_tiling_on_sc=False, needs_layout_passes=False
    ),
)
def _sc_embedding_lookup(idx_hbm, table_hbm, out_hbm, idx_v, rows_v, tile_v,
                         gsem0, gsem1, ssem0, ssem1):
    ssems = (ssem0, ssem1)
    wid = lax.axis_index("s") * _NC + lax.axis_index("c")
    gbase = wid * _NGW

    pltpu.sync_copy(idx_hbm.at[wid], idx_v)

    iota16 = lax.iota(jnp.int32, 16)
    l_vecs = [iota16 + 16 * lb for lb in range(_G // _LANES)]

    def clamp_chunk(c):
        for j in range(_KB):
            for i in range(_G // _LANES):
                sl = pl.ds(i * _LANES, _LANES)
                v = idx_v[c * _KB + j, sl]
                idx_v[c * _KB + j, sl] = jnp.minimum(
                    jnp.maximum(v, 0), _NUM_EMBEDDINGS - 1
                )

    def fire_gathers(c, h, gsem):
        clamp_chunk(c)
        for j in range(_KB):
            pltpu.async_copy(
                table_hbm.at[idx_v.at[c * _KB + j]], rows_v.at[h, j], gsem
            )

    def drain_gathers(c, h, gsem):
        for j in range(_KB):
            pltpu.make_async_copy(
                table_hbm.at[idx_v.at[c * _KB + j]], rows_v.at[h, j], gsem
            ).wait()

    def drain_one_store_group(sem):
        pltpu.make_async_copy(
            out_hbm.at[0, 0, pl.ds(0, _DB)], tile_v.at[0], sem
        ).wait()

    def process_chunk(c, h):
        for j in range(_KB):
            m = c * _KB + j
            gl = gbase + m
            sg = gl // _NB
            bb = gl % _NB
            sem = ssems[j]

            @pl.when(c >= 1)
            def _():
                drain_one_store_group(sem)

            def shift_k(k, _):
                rot = lax.rem(iota16 + k, 16)
                cols = [rot + 16 * sc for sc in range(_DIM // _LANES)]
                a_idx = [
                    lax.shift_right_logical(rot, 3) + 2 * sc
                    for sc in range(_DIM // _LANES)
                ]
                c_idx = lax.bitwise_and(rot, 7)
                for lb in range(_G // _LANES):
                    vs = [
                        plsc.load_gather(
                            rows_v.at[h, j], [l_vecs[lb], cols[sc]]
                        )
                        for sc in range(_DIM // _LANES)
                    ]
                    for sc in range(_DIM // _LANES):
                        plsc.store_scatter(
                            tile_v.at[j],
                            [a_idx[sc], c_idx, l_vecs[lb]],
                            vs[sc],
                        )
                return 0

            lax.fori_loop(0, _LANES, shift_k, 0)
            for a in range(_DB):
                pltpu.async_copy(
                    tile_v.at[j].at[a], out_hbm.at[sg].at[a].at[bb], sem
                )

    fire_gathers(0, 0, gsem0)

    def pair(t, _):
        c0 = 2 * t
        c1 = 2 * t + 1
        fire_gathers(c1, 1, gsem1)
        drain_gathers(c0, 0, gsem0)
        process_chunk(c0, 0)

        @pl.when(t + 1 < _NPAIR)
        def _():
            fire_gathers(c0 + 2, 0, gsem0)

        drain_gathers(c1, 1, gsem1)
        process_chunk(c1, 1)
        return 0

    lax.fori_loop(0, _NPAIR, pair, 0)

    for j in range(_KB):
        drain_one_store_group(ssems[j])


def kernel(idx, table):
    flat = idx.T.reshape(_NW, _NGW, _G).astype(jnp.int32)
    tab_pad = jnp.pad(table, ((0, 0), (0, _DIM)))
    out = _sc_embedding_lookup(flat, tab_pad)
    return out.transpose(2, 4, 0, 1, 3).reshape(_B, _S, _DIM)
